# speedup vs baseline: 7.3366x; 7.3366x over previous
import functools

import jax
import jax.numpy as jnp
from jax import lax
from jax.experimental import pallas as pl
from jax.experimental.pallas import tpu as pltpu
from jax.experimental.pallas import tpu_sc as plsc

VOCAB = 100000
HIDDEN = 128
OUT = 1000
B = 4096
L = 50

NC = 2
NS = 16
LANES = 16
NW = NC * NS
B_PER_W = B // NW
CHUNK_S = 2
ROWS_PER_CHUNK = CHUNK_S * L
NCHUNK = B_PER_W // CHUNK_S
NVEC = HIDDEN // LANES


def _bag_body(idx_hbm, table_hbm, out_hbm, idx_v, rows_v, pool_v, sem_g):
    wid = lax.axis_index("s") * NC + lax.axis_index("c")

    pltpu.sync_copy(idx_hbm.at[pl.ds(wid * NCHUNK, NCHUNK)], idx_v)

    def start_gather(j, buf):
        pltpu.async_copy(table_hbm.at[idx_v.at[j]], rows_v.at[buf], sem_g)

    def wait_gather(j, buf):
        pltpu.make_async_copy(table_hbm.at[idx_v.at[j]], rows_v.at[buf],
                              sem_g).wait()

    start_gather(0, 0)

    def accum_chunk(j, buf):
        def row_body(l, accs):
            new = []
            for s in range(CHUNK_S):
                r = s * L + l
                for h in range(NVEC):
                    new.append(accs[s * NVEC + h]
                               + rows_v[buf, r, pl.ds(h * LANES, LANES)])
            return tuple(new)

        zeros = tuple(jnp.zeros((LANES,), jnp.float32)
                      for _ in range(CHUNK_S * NVEC))
        accs = lax.fori_loop(0, L, row_body, zeros)
        for s in range(CHUNK_S):
            for h in range(NVEC):
                pool_v[j * CHUNK_S + s, pl.ds(h * LANES, LANES)] = (
                    accs[s * NVEC + h])

    def outer(jj, carry):
        for b in range(2):
            j = jj * 2 + b
            wait_gather(j, b)

            @pl.when(j + 1 < NCHUNK)
            def _():
                start_gather(j + 1, 1 - b)

            accum_chunk(j, b)
        return carry

    lax.fori_loop(0, NCHUNK // 2, outer, 0)

    pltpu.sync_copy(pool_v, out_hbm.at[pl.ds(wid * B_PER_W, B_PER_W)])


_bag = functools.partial(
    pl.kernel,
    out_type=jax.ShapeDtypeStruct((B, HIDDEN), jnp.float32),
    mesh=plsc.VectorSubcoreMesh(core_axis_name="c", subcore_axis_name="s"),
    scratch_types=[
        pltpu.VMEM((NCHUNK, ROWS_PER_CHUNK), jnp.int32),
        pltpu.VMEM((2, ROWS_PER_CHUNK, HIDDEN), jnp.float32),
        pltpu.VMEM((B_PER_W, HIDDEN), jnp.float32),
        pltpu.SemaphoreType.DMA,
    ],
)(_bag_body)


def _mm_body(x_ref, w_ref, b_ref, o_ref):
    o_ref[...] = (
        jnp.dot(x_ref[...], w_ref[...], preferred_element_type=jnp.float32)
        * (1.0 / L)
        + b_ref[...])


def _matmul(pooled, fc_w, fc_b2):
    bm = 512
    return pl.pallas_call(
        _mm_body,
        grid=(B // bm,),
        in_specs=[
            pl.BlockSpec((bm, HIDDEN), lambda i: (i, 0)),
            pl.BlockSpec((HIDDEN, OUT), lambda i: (0, 0)),
            pl.BlockSpec((1, OUT), lambda i: (0, 0)),
        ],
        out_specs=pl.BlockSpec((bm, OUT), lambda i: (i, 0)),
        out_shape=jax.ShapeDtypeStruct((B, OUT), jnp.float32),
    )(pooled, fc_w, fc_b2)


def kernel(x, emb_table, fc_w, fc_b):
    idx = x.reshape(NW * NCHUNK, ROWS_PER_CHUNK).astype(jnp.int32)
    pooled = _bag(idx, emb_table)
    return _matmul(pooled, fc_w, fc_b.reshape(1, OUT))


# speedup vs baseline: 13.4144x; 1.8284x over previous
import functools

import jax
import jax.numpy as jnp
from jax import lax
from jax.experimental import pallas as pl
from jax.experimental.pallas import tpu as pltpu
from jax.experimental.pallas import tpu_sc as plsc

VOCAB = 100000
HIDDEN = 128
OUT = 1000
B = 4096
L = 50

NC = 2
NS = 16
LANES = 16
NW = NC * NS
B_PER_W = B // NW
CHUNK_S = 2
ROWS_PER_CHUNK = CHUNK_S * L
NCHUNK = B_PER_W // CHUNK_S
NVEC = HIDDEN // LANES
NBUF = 4


def _bag_body(idx_hbm, table_hbm, out_hbm, idx_v, rows_v, pool_v,
              sem0, sem1, sem2, sem3):
    sems = (sem0, sem1, sem2, sem3)
    wid = lax.axis_index("s") * NC + lax.axis_index("c")

    pltpu.sync_copy(idx_hbm.at[pl.ds(wid * NCHUNK, NCHUNK)], idx_v)

    def start_gather(j, buf):
        pltpu.async_copy(table_hbm.at[idx_v.at[j]], rows_v.at[buf], sems[buf])

    def wait_gather(j, buf):
        pltpu.make_async_copy(table_hbm.at[idx_v.at[j]], rows_v.at[buf],
                              sems[buf]).wait()

    for b in range(NBUF - 1):
        start_gather(b, b)

    def accum_chunk(j, buf):
        def row_body(l, accs):
            new = list(accs)
            for u in range(2):
                for s in range(CHUNK_S):
                    r = s * L + 2 * l + u
                    for h in range(NVEC):
                        k = s * NVEC + h
                        new[k] = new[k] + rows_v[buf, r, pl.ds(h * LANES,
                                                               LANES)]
            return tuple(new)

        zeros = tuple(jnp.zeros((LANES,), jnp.float32)
                      for _ in range(CHUNK_S * NVEC))
        accs = lax.fori_loop(0, L // 2, row_body, zeros)
        for s in range(CHUNK_S):
            for h in range(NVEC):
                pool_v[j * CHUNK_S + s, pl.ds(h * LANES, LANES)] = (
                    accs[s * NVEC + h])

    def outer(jj, carry):
        for b in range(NBUF):
            j = jj * NBUF + b
            wait_gather(j, b)

            @pl.when(j + NBUF - 1 < NCHUNK)
            def _():
                start_gather(j + NBUF - 1, (b + NBUF - 1) % NBUF)

            accum_chunk(j, b)
        return carry

    lax.fori_loop(0, NCHUNK // NBUF, outer, 0)

    pltpu.sync_copy(pool_v, out_hbm.at[pl.ds(wid * B_PER_W, B_PER_W)])


_bag = functools.partial(
    pl.kernel,
    out_type=jax.ShapeDtypeStruct((B, HIDDEN), jnp.float32),
    mesh=plsc.VectorSubcoreMesh(core_axis_name="c", subcore_axis_name="s"),
    scratch_types=[
        pltpu.VMEM((NCHUNK, ROWS_PER_CHUNK), jnp.int32),
        pltpu.VMEM((NBUF, ROWS_PER_CHUNK, HIDDEN), jnp.float32),
        pltpu.VMEM((B_PER_W, HIDDEN), jnp.float32),
        pltpu.SemaphoreType.DMA,
        pltpu.SemaphoreType.DMA,
        pltpu.SemaphoreType.DMA,
        pltpu.SemaphoreType.DMA,
    ],
)(_bag_body)


def _mm_body(w_ref, x_ref, b_ref, o_ref):
    o_ref[...] = (
        lax.dot_general(w_ref[...], x_ref[...],
                        dimension_numbers=(((0,), (1,)), ((), ())),
                        preferred_element_type=jnp.float32)
        * (1.0 / L)
        + b_ref[...])


def _matmul_t(pooled, fc_w, fc_bt):
    bm = 512
    return pl.pallas_call(
        _mm_body,
        grid=(B // bm,),
        in_specs=[
            pl.BlockSpec((HIDDEN, OUT), lambda i: (0, 0)),
            pl.BlockSpec((bm, HIDDEN), lambda i: (i, 0)),
            pl.BlockSpec((OUT, 1), lambda i: (0, 0)),
        ],
        out_specs=pl.BlockSpec((OUT, bm), lambda i: (0, i)),
        out_shape=jax.ShapeDtypeStruct((OUT, B), jnp.float32),
    )(fc_w, pooled, fc_bt)


def kernel(x, emb_table, fc_w, fc_b):
    idx = x.reshape(NW * NCHUNK, ROWS_PER_CHUNK).astype(jnp.int32)
    pooled = _bag(idx, emb_table)
    out_t = _matmul_t(pooled, fc_w, fc_b.reshape(OUT, 1))
    return out_t.T


# speedup vs baseline: 13.6270x; 1.0158x over previous
import functools

import jax
import jax.numpy as jnp
from jax import lax
from jax.experimental import pallas as pl
from jax.experimental.pallas import tpu as pltpu
from jax.experimental.pallas import tpu_sc as plsc

VOCAB = 100000
HIDDEN = 128
OUT = 1000
B = 4096
L = 50

NC = 2
NS = 16
LANES = 16
NW = NC * NS
B_PER_W = B // NW
NVEC = HIDDEN // LANES
NBUF = 8


def _bag_body(idx_hbm, table_hbm, out_hbm, idx_v, rows_v, pool_v, *sems):
    wid = lax.axis_index("s") * NC + lax.axis_index("c")

    pltpu.sync_copy(idx_hbm.at[pl.ds(wid * B_PER_W, B_PER_W)], idx_v)

    def start_gather(s, buf):
        pltpu.async_copy(table_hbm.at[idx_v.at[s]], rows_v.at[buf], sems[buf])

    def wait_gather(s, buf):
        pltpu.make_async_copy(table_hbm.at[idx_v.at[s]], rows_v.at[buf],
                              sems[buf]).wait()

    for b in range(NBUF - 1):
        start_gather(b, b)

    def accum_sample(s, buf):
        def row_body(l, accs):
            new = list(accs)
            for u in range(2):
                r = 2 * l + u
                for h in range(NVEC):
                    new[h] = new[h] + rows_v[buf, r, pl.ds(h * LANES, LANES)]
            return tuple(new)

        zeros = tuple(jnp.zeros((LANES,), jnp.float32) for _ in range(NVEC))
        accs = lax.fori_loop(0, L // 2, row_body, zeros)
        for h in range(NVEC):
            pool_v[s, pl.ds(h * LANES, LANES)] = accs[h]

    def outer(jj, carry):
        for b in range(NBUF):
            s = jj * NBUF + b
            wait_gather(s, b)

            @pl.when(s + NBUF - 1 < B_PER_W)
            def _():
                start_gather(s + NBUF - 1, (b + NBUF - 1) % NBUF)

            accum_sample(s, b)
        return carry

    lax.fori_loop(0, B_PER_W // NBUF, outer, 0)

    pltpu.sync_copy(pool_v, out_hbm.at[pl.ds(wid * B_PER_W, B_PER_W)])


_bag = functools.partial(
    pl.kernel,
    out_type=jax.ShapeDtypeStruct((B, HIDDEN), jnp.float32),
    mesh=plsc.VectorSubcoreMesh(core_axis_name="c", subcore_axis_name="s"),
    scratch_types=[
        pltpu.VMEM((B_PER_W, L), jnp.int32),
        pltpu.VMEM((NBUF, L, HIDDEN), jnp.float32),
        pltpu.VMEM((B_PER_W, HIDDEN), jnp.float32),
    ] + [pltpu.SemaphoreType.DMA] * NBUF,
)(_bag_body)


def _mm_body(w_ref, x_ref, b_ref, o_ref):
    o_ref[...] = (
        lax.dot_general(w_ref[...], x_ref[...],
                        dimension_numbers=(((0,), (1,)), ((), ())),
                        preferred_element_type=jnp.float32)
        * (1.0 / L)
        + b_ref[...])


def _matmul_t(pooled, fc_w, fc_bt):
    bm = 512
    return pl.pallas_call(
        _mm_body,
        grid=(B // bm,),
        in_specs=[
            pl.BlockSpec((HIDDEN, OUT), lambda i: (0, 0)),
            pl.BlockSpec((bm, HIDDEN), lambda i: (i, 0)),
            pl.BlockSpec((OUT, 1), lambda i: (0, 0)),
        ],
        out_specs=pl.BlockSpec((OUT, bm), lambda i: (0, i)),
        out_shape=jax.ShapeDtypeStruct((OUT, B), jnp.float32),
    )(fc_w, pooled, fc_bt)


def kernel(x, emb_table, fc_w, fc_b):
    pooled = _bag(x.astype(jnp.int32), emb_table)
    out_t = _matmul_t(pooled, fc_w, fc_b.reshape(OUT, 1))
    return out_t.T


# speedup vs baseline: 13.6741x; 1.0035x over previous
import functools

import jax
import jax.numpy as jnp
from jax import lax
from jax.experimental import pallas as pl
from jax.experimental.pallas import tpu as pltpu
from jax.experimental.pallas import tpu_sc as plsc

VOCAB = 100000
HIDDEN = 128
OUT = 1000
B = 4096
L = 50

NC = 2
NS = 16
LANES = 16
NW = NC * NS
B_PER_W = B // NW
NVEC = HIDDEN // LANES
NBUF = 8


def _bag_body(idx_hbm, table_hbm, out_hbm, idx_v, rows_v, pool_v, *sems):
    wid = lax.axis_index("s") * NC + lax.axis_index("c")

    pltpu.sync_copy(idx_hbm.at[pl.ds(wid * B_PER_W, B_PER_W)], idx_v)

    def start_gather(s, buf):
        pltpu.async_copy(table_hbm.at[idx_v.at[s]], rows_v.at[buf], sems[buf])

    def wait_gather(s, buf):
        pltpu.make_async_copy(table_hbm.at[idx_v.at[s]], rows_v.at[buf],
                              sems[buf]).wait()

    for b in range(NBUF - 1):
        start_gather(b, b)

    def accum_sample(s, buf):
        def row_body(l, accs):
            new = list(accs)
            for u in range(2):
                r = 2 * l + u
                for h in range(NVEC):
                    new[h] = new[h] + rows_v[buf, r, pl.ds(h * LANES, LANES)]
            return tuple(new)

        zeros = tuple(jnp.zeros((LANES,), jnp.float32) for _ in range(NVEC))
        accs = lax.fori_loop(0, L // 2, row_body, zeros)
        for h in range(NVEC):
            pool_v[s, pl.ds(h * LANES, LANES)] = accs[h]

    def outer(jj, carry):
        for b in range(NBUF):
            s = jj * NBUF + b
            wait_gather(s, b)

            @pl.when(s + NBUF - 1 < B_PER_W)
            def _():
                start_gather(s + NBUF - 1, (b + NBUF - 1) % NBUF)

            accum_sample(s, b)
        return carry

    lax.fori_loop(0, B_PER_W // NBUF, outer, 0)

    pltpu.sync_copy(pool_v, out_hbm.at[pl.ds(wid * B_PER_W, B_PER_W)])


_bag = functools.partial(
    pl.kernel,
    out_type=jax.ShapeDtypeStruct((B, HIDDEN), jnp.float32),
    mesh=plsc.VectorSubcoreMesh(core_axis_name="c", subcore_axis_name="s"),
    scratch_types=[
        pltpu.VMEM((B_PER_W, L), jnp.int32),
        pltpu.VMEM((NBUF, L, HIDDEN), jnp.float32),
        pltpu.VMEM((B_PER_W, HIDDEN), jnp.float32),
    ] + [pltpu.SemaphoreType.DMA] * NBUF,
)(_bag_body)


def _mm_body(wt_ref, x_ref, b_ref, o_ref):
    o_ref[...] = (
        lax.dot_general(wt_ref[...], x_ref[...],
                        dimension_numbers=(((1,), (1,)), ((), ())),
                        preferred_element_type=jnp.float32)
        * (1.0 / L)
        + b_ref[...].T)


def _matmul_t(pooled, fc_wt, fc_b2):
    bm = 512
    return pl.pallas_call(
        _mm_body,
        grid=(B // bm,),
        in_specs=[
            pl.BlockSpec((OUT, HIDDEN), lambda i: (0, 0)),
            pl.BlockSpec((bm, HIDDEN), lambda i: (i, 0)),
            pl.BlockSpec((1, OUT), lambda i: (0, 0)),
        ],
        out_specs=pl.BlockSpec((OUT, bm), lambda i: (0, i)),
        out_shape=jax.ShapeDtypeStruct((OUT, B), jnp.float32),
    )(fc_wt, pooled, fc_b2)


def kernel(x, emb_table, fc_w, fc_b):
    pooled = _bag(x.astype(jnp.int32), emb_table)
    out_t = _matmul_t(pooled, fc_w.T, fc_b.reshape(1, OUT))
    return out_t.T
